# Initial kernel scaffold; baseline (speedup 1.0000x reference)
import jax
import jax.numpy as jnp
from jax.experimental import pallas as pl


def kernel(det_rois, det_labels, det_scores, im_info):
    raise NotImplementedError("write your pallas kernel here")


# speedup vs baseline: 9.4880x; 9.4880x over previous
import functools

import jax
import jax.numpy as jnp
from jax.experimental import pallas as pl
from jax.experimental.pallas import tpu as pltpu

_R = 8


def _body(pars_ref, rois_ref, roisT_ref, labr_ref, labc_ref, scr_ref, scc_ref,
          sbjr_ref, objr_ref, relr_ref, slab_ref, olab_ref, ssc_ref, osc_ref,
          spt_ref, *, n):
    m = n - 1
    r = _R
    i = pl.program_id(0)
    r0 = i * r

    inv_scale = pars_ref[0, 0]
    rw = pars_ref[0, 1]
    rh = pars_ref[0, 2]
    rwh = pars_ref[0, 3]

    rows = r0 + jax.lax.broadcasted_iota(jnp.int32, (r, 1), 0)
    cit = jax.lax.broadcasted_iota(jnp.int32, (r, m), 1)
    mask = cit < rows

    def sel(v):
        return jnp.where(mask, v[:, :m], v[:, 1:n])

    rois_blk = rois_ref[pl.ds(r0, r), :]

    sbj3 = jnp.broadcast_to(rois_blk.reshape(r, 1, 5), (r, m, 5))
    a3 = rois_ref[0:m, :].reshape(1, m, 5)
    b3 = rois_ref[1:n, :].reshape(1, m, 5)
    mask3 = jax.lax.broadcasted_iota(jnp.int32, (r, m, 1), 1) < rows.reshape(r, 1, 1)
    obj3 = jnp.where(mask3, a3, b3)
    sbjr_ref[...] = sbj3
    objr_ref[...] = obj3
    relr_ref[...] = jnp.concatenate([
        sbj3[:, :, 0:1],
        jnp.minimum(sbj3[:, :, 1:3], obj3[:, :, 1:3]),
        jnp.maximum(sbj3[:, :, 3:5], obj3[:, :, 3:5])], axis=2)

    slab_ref[...] = jnp.broadcast_to(labc_ref[pl.ds(r0, r), :], (r, m))
    olab_ref[...] = sel(labr_ref[...])
    ssc_ref[...] = jnp.broadcast_to(scc_ref[pl.ds(r0, r), :], (r, m))
    osc_ref[...] = sel(scr_ref[...])

    sx1 = rois_blk[:, 1:2] * inv_scale
    sy1 = rois_blk[:, 2:3] * inv_scale
    sx2 = rois_blk[:, 3:4] * inv_scale
    sy2 = rois_blk[:, 4:5] * inv_scale
    ox1 = sel(roisT_ref[1:2, :]) * inv_scale
    oy1 = sel(roisT_ref[2:3, :]) * inv_scale
    ox2 = sel(roisT_ref[3:4, :]) * inv_scale
    oy2 = sel(roisT_ref[4:5, :]) * inv_scale

    sw = sx2 - sx1 + 1.0
    sh = sy2 - sy1 + 1.0
    scx = sx1 + 0.5 * sw
    scy = sy1 + 0.5 * sh
    ow = ox2 - ox1 + 1.0
    oh = oy2 - oy1 + 1.0
    ocx = ox1 + 0.5 * ow
    ocy = oy1 + 0.5 * oh
    ux1 = jnp.minimum(sx1, ox1)
    uy1 = jnp.minimum(sy1, oy1)
    ux2 = jnp.maximum(sx2, ox2)
    uy2 = jnp.maximum(sy2, oy2)
    uw = ux2 - ux1 + 1.0
    uh = uy2 - uy1 + 1.0
    ucx = ux1 + 0.5 * uw
    ucy = uy1 + 0.5 * uh

    rsw = 1.0 / sw
    rsh = 1.0 / sh
    row_ = 1.0 / ow
    roh = 1.0 / oh
    ruw = 1.0 / uw
    ruh = 1.0 / uh
    lsw = jnp.log(sw)
    lsh = jnp.log(sh)
    low = jnp.log(ow)
    loh = jnp.log(oh)
    luw = jnp.log(uw)
    luh = jnp.log(uh)

    feats = [
        (ocx - scx) * rsw, (ocy - scy) * rsh, low - lsw, loh - lsh,
        (scx - ocx) * row_, (scy - ocy) * roh,
        (ucx - scx) * rsw, (ucy - scy) * rsh, luw - lsw, luh - lsh,
        (scx - ucx) * ruw, (scy - ucy) * ruh,
        (ocx - ucx) * ruw, (ocy - ucy) * ruh, low - luw, loh - luh,
        (ucx - ocx) * row_, (ucy - ocy) * roh,
        sx1 * rw, sy1 * rh, sx2 * rw, sy2 * rh, sw * sh * rwh,
        ox1 * rw, oy1 * rh, ox2 * rw, oy2 * rh, ow * oh * rwh,
    ]
    feats = [jnp.broadcast_to(f, (r, m)) for f in feats]
    stk = jnp.stack(feats, axis=1)
    spt_ref[...] = jnp.transpose(stk, (0, 2, 1))


def kernel(det_rois, det_labels, det_scores, im_info):
    n = det_rois.shape[0]
    m = n - 1
    r = _R
    grid = (n // r,)
    f32 = jnp.float32
    ldt = det_labels.dtype

    rois_t = det_rois.T
    labr = det_labels.reshape(1, n)
    labc = det_labels.reshape(n, 1)
    scr = det_scores.reshape(1, n)
    scc = det_scores.reshape(n, 1)
    im_h = im_info[0, 0]
    im_w = im_info[0, 1]
    pars = jnp.stack([1.0 / im_info[0, 2], 1.0 / im_w, 1.0 / im_h,
                      1.0 / (im_w * im_h)]).reshape(1, 4).astype(f32)

    full = lambda shape: pl.BlockSpec(shape, lambda i: tuple(0 for _ in shape))
    out = pl.pallas_call(
        functools.partial(_body, n=n),
        grid=grid,
        in_specs=[
            pl.BlockSpec(memory_space=pltpu.SMEM),
            full((n, 5)), full((5, n)),
            full((1, n)), full((n, 1)),
            full((1, n)), full((n, 1)),
        ],
        out_specs=[
            pl.BlockSpec((r, m, 5), lambda i: (i, 0, 0)),
            pl.BlockSpec((r, m, 5), lambda i: (i, 0, 0)),
            pl.BlockSpec((r, m, 5), lambda i: (i, 0, 0)),
            pl.BlockSpec((r, m), lambda i: (i, 0)),
            pl.BlockSpec((r, m), lambda i: (i, 0)),
            pl.BlockSpec((r, m), lambda i: (i, 0)),
            pl.BlockSpec((r, m), lambda i: (i, 0)),
            pl.BlockSpec((r, m, 28), lambda i: (i, 0, 0)),
        ],
        out_shape=[
            jax.ShapeDtypeStruct((n, m, 5), f32),
            jax.ShapeDtypeStruct((n, m, 5), f32),
            jax.ShapeDtypeStruct((n, m, 5), f32),
            jax.ShapeDtypeStruct((n, m), ldt),
            jax.ShapeDtypeStruct((n, m), ldt),
            jax.ShapeDtypeStruct((n, m), f32),
            jax.ShapeDtypeStruct((n, m), f32),
            jax.ShapeDtypeStruct((n, m, 28), f32),
        ],
    )(pars, det_rois, rois_t, labr, labc, scr, scc)
    sbj_rois, obj_rois, rel_rois, slab, olab, ssc, osc, spt = out
    nm = n * m
    return (det_rois,
            sbj_rois.reshape(nm, 5),
            obj_rois.reshape(nm, 5),
            rel_rois.reshape(nm, 5),
            slab.reshape(nm),
            olab.reshape(nm),
            ssc.reshape(nm),
            osc.reshape(nm),
            spt.reshape(nm, 28),
            jnp.array([nm], dtype=jnp.int32))
